# baseline (speedup 1.0000x reference)
import functools

import jax
import jax.numpy as jnp
from jax import lax
from jax.experimental import pallas as pl
from jax.experimental.pallas import tpu as pltpu
from jax.experimental.pallas import tpu_sc as plsc

_EPS = 1e-5
_BN = 1000

_NC = 2
_NS = 16
_W = 32
_NWIN = 320
_EPAD = _NS * _NWIN * _W
_N = 10000
_NACC = 10240
_RPT = _NACC // _NS
_HF = 128


def _in_proj_kernel(x_ref, w_ref, b_ref, o_ref):
    acc = lax.dot_general(x_ref[...], w_ref[...], (((1,), (1,)), ((), ())),
                          preferred_element_type=jnp.float32)
    o_ref[...] = jnp.maximum(acc + b_ref[...], 0.0)


def _layer_kernel(a0_ref, a1_ref, cnt_ref, h_ref, wl_ref, bl_ref, wr_ref,
                  g_ref, be_ref, o_ref):
    inv = 1.0 / jnp.maximum(cnt_ref[...][:, :1], 1.0)
    h = h_ref[...]
    wl = wl_ref[...]
    t = lax.dot_general(a0_ref[...] * inv, wl[:, :_HF],
                        (((1,), (1,)), ((), ())),
                        preferred_element_type=jnp.float32)
    t += lax.dot_general(a1_ref[...] * inv, wl[:, _HF:],
                         (((1,), (1,)), ((), ())),
                         preferred_element_type=jnp.float32)
    t += lax.dot_general(h, wr_ref[...], (((1,), (1,)), ((), ())),
                         preferred_element_type=jnp.float32)
    t += bl_ref[...]
    mu = jnp.mean(t, axis=-1, keepdims=True)
    d = t - mu
    var = jnp.mean(d * d, axis=-1, keepdims=True)
    t = d * lax.rsqrt(var + _EPS) * g_ref[...] + be_ref[...]
    o_ref[...] = jnp.maximum(t, 0.0) + h


def _out_proj_kernel(h_ref, w_ref, b_ref, o_ref):
    o_ref[...] = jnp.sum(h_ref[...] * w_ref[...], axis=-1,
                         keepdims=True) + b_ref[...]


_CH = 32
_NCHUNK = _NWIN // _CH
_ND = 4


def _agg_body(h2, srcr, dstr, out, acc, src_c, dst_c, gb0, gb1, gb2, gb3,
              gs0, gs1, gs2, gs3, ss0, ss1, ss2, ss3, is0, is1):
    c = lax.axis_index("c")
    s = lax.axis_index("s")
    isems = (is0, is1)
    gsems = (gs0, gs1, gs2, gs3)
    ssems = (ss0, ss1, ss2, ss3)
    gbufs = (gb0, gb1, gb2, gb3)

    z16 = jnp.zeros((16,), jnp.float32)

    def zrow(i, _):
        for j in range(8):
            gb0[i, pl.ds(j * 16, 16)] = z16
        return 0
    lax.fori_loop(0, _W, zrow, 0, unroll=2)
    zbase = s * (_NACC // _NS)
    nz = _NACC // _NS // _W
    for k in range(nz):
        pltpu.async_copy(gb0, acc.at[pl.ds(zbase + k * _W, _W)],
                         gsems[k % _ND])
    for k in range(nz):
        pltpu.make_async_copy(gb0, acc.at[pl.ds(zbase + k * _W, _W)],
                              gsems[k % _ND]).wait()

    plsc.subcore_barrier()

    def fetch_chunk(k, p):
        pltpu.async_copy(srcr.at[s].at[pl.ds(k * _CH, _CH)],
                         src_c.at[p], isems[p])
        pltpu.async_copy(dstr.at[s].at[pl.ds(k * _CH, _CH)],
                         dst_c.at[p], isems[p])

    fetch_chunk(0, 0)
    for k in range(_NCHUNK):
        p = k % 2
        scp = src_c.at[p]
        dcp = dst_c.at[p]
        pltpu.make_async_copy(srcr.at[s].at[pl.ds(k * _CH, _CH)],
                              scp, isems[p]).wait()
        pltpu.make_async_copy(dstr.at[s].at[pl.ds(k * _CH, _CH)],
                              dcp, isems[p]).wait()

        def irow(w, _):
            for j in range(_W // 16):
                v = scp[w, pl.ds(j * 16, 16)]
                scp[w, pl.ds(j * 16, 16)] = v * 2 + c
            return 0
        lax.fori_loop(0, _CH, irow, 0, unroll=2)

        if k + 1 < _NCHUNK:
            fetch_chunk(k + 1, 1 - p)

        pltpu.async_copy(h2.at[scp.at[0]], gb0, gs0)
        pltpu.async_copy(h2.at[scp.at[1]], gb1, gs1)

        def step(i, _):
            g = i * _ND
            for b in range(_ND):
                w = g + b
                gbuf, gsem, ssem = gbufs[b], gsems[b], ssems[b]
                pltpu.make_async_copy(h2.at[scp.at[w]], gbuf, gsem).wait()
                pltpu.async_copy(gbuf, acc.at[dcp.at[w]], ssem, add=True)

                @pl.when(w >= 2)
                def _():
                    b2 = (b + 2) % _ND
                    pltpu.make_async_copy(gbufs[b2], acc.at[dcp.at[w - 2]],
                                          ssems[b2]).wait()

                @pl.when(w + 2 < _CH)
                def _():
                    b2 = (b + 2) % _ND
                    pltpu.async_copy(h2.at[scp.at[w + 2]], gbufs[b2],
                                     gsems[b2])
            return 0
        lax.fori_loop(0, _CH // _ND, step, 0)

        for w in (_CH - 2, _CH - 1):
            b = w % _ND
            pltpu.make_async_copy(gbufs[b], acc.at[dcp.at[w]],
                                  ssems[b]).wait()

    plsc.subcore_barrier()

    rbase = s * _RPT
    pltpu.sync_copy(acc.at[pl.ds(rbase, _RPT)],
                    out.at[c].at[pl.ds(rbase, _RPT)])


def _build_agg():
    mesh = plsc.VectorSubcoreMesh(core_axis_name="c", subcore_axis_name="s")
    return pl.kernel(
        _agg_body,
        out_type=jax.ShapeDtypeStruct((_NC, _NACC, _HF), jnp.float32),
        mesh=mesh,
        scratch_types=[
            pltpu.VMEM_SHARED((_NACC, _HF), jnp.float32),
            pltpu.VMEM((2, _CH, _W), jnp.int32),
            pltpu.VMEM((2, _CH, _W), jnp.int32),
            pltpu.VMEM((_W, _HF), jnp.float32),
            pltpu.VMEM((_W, _HF), jnp.float32),
            pltpu.VMEM((_W, _HF), jnp.float32),
            pltpu.VMEM((_W, _HF), jnp.float32),
        ] + [pltpu.SemaphoreType.DMA] * 10)


_agg = _build_agg()


def _row_blocked(kern, n, h, extra_specs, out_cols):
    return pl.pallas_call(
        kern,
        grid=(n // _BN,),
        in_specs=[pl.BlockSpec((_BN, h), lambda i: (i, 0))] + extra_specs,
        out_specs=pl.BlockSpec((_BN, out_cols), lambda i: (i, 0)),
        out_shape=jax.ShapeDtypeStruct((n, out_cols), jnp.float32),
    )


def kernel(x, edge_index, W_in, b_in, Wl, bl, Wr, gamma, beta, W_out, b_out):
    n, in_dim = x.shape
    h_dim = W_in.shape[0]
    L = Wl.shape[0]
    e = edge_index.shape[1]
    src = edge_index[0]
    dst = edge_index[1]

    pad = _EPAD - e
    srcr = jnp.concatenate([src, jnp.zeros((pad,), jnp.int32)]
                           ).reshape(_NS, _NWIN, _W)
    dstr = jnp.concatenate([dst, jnp.full((pad,), _N, jnp.int32)]
                           ).reshape(_NS, _NWIN, _W)

    wspec = pl.BlockSpec((h_dim, in_dim), lambda i: (0, 0))
    vspec = pl.BlockSpec((1, h_dim), lambda i: (0, 0))

    h = _row_blocked(_in_proj_kernel, n, in_dim,
                     [wspec, vspec], h_dim)(x, W_in, b_in.reshape(1, h_dim))

    layer = pl.pallas_call(
        _layer_kernel,
        grid=(n // _BN,),
        in_specs=[
            pl.BlockSpec((_BN, _HF), lambda i: (i, 0)),
            pl.BlockSpec((_BN, _HF), lambda i: (i, 0)),
            pl.BlockSpec((_BN, _HF), lambda i: (i, 0)),
            pl.BlockSpec((_BN, h_dim), lambda i: (i, 0)),
            pl.BlockSpec((h_dim, h_dim), lambda i: (0, 0)),
            vspec,
            pl.BlockSpec((h_dim, h_dim), lambda i: (0, 0)),
            vspec, vspec,
        ],
        out_specs=pl.BlockSpec((_BN, h_dim), lambda i: (i, 0)),
        out_shape=jax.ShapeDtypeStruct((n, h_dim), jnp.float32),
    )

    cnt16 = _agg(jnp.ones((2 * n, _HF), jnp.float32), srcr, dstr)[0]
    for i in range(L):
        h2 = h.reshape(2 * n, _HF)
        aggs = _agg(h2, srcr, dstr)
        h = layer(aggs[0], aggs[1], cnt16, h, Wl[i],
                  bl[i].reshape(1, h_dim), Wr[i],
                  gamma[i].reshape(1, h_dim), beta[i].reshape(1, h_dim))

    out = _row_blocked(_out_proj_kernel, n, h_dim,
                       [vspec, pl.BlockSpec((1, 1), lambda i: (0, 0))], 1)(
        h, W_out.reshape(1, h_dim), b_out.reshape(1, 1))
    return out.reshape(-1)


# speedup vs baseline: 1.2885x; 1.2885x over previous
import functools

import jax
import jax.numpy as jnp
from jax import lax
from jax.experimental import pallas as pl
from jax.experimental.pallas import tpu as pltpu
from jax.experimental.pallas import tpu_sc as plsc

_EPS = 1e-5
_BN = 1000

_NC = 2
_NS = 16
_W = 64
_NWIN = 160
_EPAD = _NS * _NWIN * _W
_N = 10000
_NACC = 10240
_RPT = _NACC // _NS
_HF = 128


def _in_proj_kernel(x_ref, w_ref, b_ref, o_ref):
    acc = lax.dot_general(x_ref[...], w_ref[...], (((1,), (1,)), ((), ())),
                          preferred_element_type=jnp.float32)
    o_ref[...] = jnp.maximum(acc + b_ref[...], 0.0)


def _layer_kernel(a0_ref, a1_ref, cnt_ref, h_ref, wl_ref, bl_ref, wr_ref,
                  g_ref, be_ref, o_ref):
    inv = 1.0 / jnp.maximum(cnt_ref[...][:, :1], 1.0)
    h = h_ref[...]
    wl = wl_ref[...]
    t = lax.dot_general(a0_ref[...] * inv, wl[:, :_HF],
                        (((1,), (1,)), ((), ())),
                        preferred_element_type=jnp.float32)
    t += lax.dot_general(a1_ref[...] * inv, wl[:, _HF:],
                         (((1,), (1,)), ((), ())),
                         preferred_element_type=jnp.float32)
    t += lax.dot_general(h, wr_ref[...], (((1,), (1,)), ((), ())),
                         preferred_element_type=jnp.float32)
    t += bl_ref[...]
    mu = jnp.mean(t, axis=-1, keepdims=True)
    d = t - mu
    var = jnp.mean(d * d, axis=-1, keepdims=True)
    t = d * lax.rsqrt(var + _EPS) * g_ref[...] + be_ref[...]
    o_ref[...] = jnp.maximum(t, 0.0) + h


def _out_proj_kernel(h_ref, w_ref, b_ref, o_ref):
    o_ref[...] = jnp.sum(h_ref[...] * w_ref[...], axis=-1,
                         keepdims=True) + b_ref[...]


_CH = 16
_NCHUNK = _NWIN // _CH


def _agg_body(tw, h2, srcr, dstr, out, acc, src_c, dst_c, gb0, gb1,
              gs0, gs1, ss0, ss1, is0, is1):
    c = lax.axis_index("c")
    s = lax.axis_index("s")
    isems = (is0, is1)
    gsems = (gs0, gs1)
    ssems = (ss0, ss1)
    gbufs = (gb0, gb1)

    z16 = jnp.zeros((16,), jnp.float32)

    def zrow(i, _):
        for j in range(tw // 16):
            gb0[i, pl.ds(j * 16, 16)] = z16
        return 0
    lax.fori_loop(0, _W, zrow, 0, unroll=2)
    zbase = s * (_NACC // _NS)
    nz = _NACC // _NS // _W
    for k in range(nz):
        pltpu.async_copy(gb0, acc.at[pl.ds(zbase + k * _W, _W)],
                         gsems[k % 2])
    for k in range(nz):
        pltpu.make_async_copy(gb0, acc.at[pl.ds(zbase + k * _W, _W)],
                              gsems[k % 2]).wait()

    plsc.subcore_barrier()

    def fetch_chunk(k, p):
        pltpu.async_copy(srcr.at[s].at[pl.ds(k * _CH, _CH)],
                         src_c.at[p], isems[p])
        pltpu.async_copy(dstr.at[s].at[pl.ds(k * _CH, _CH)],
                         dst_c.at[p], isems[p])

    fetch_chunk(0, 0)
    for k in range(_NCHUNK):
        p = k % 2
        scp = src_c.at[p]
        dcp = dst_c.at[p]
        pltpu.make_async_copy(srcr.at[s].at[pl.ds(k * _CH, _CH)],
                              scp, isems[p]).wait()
        pltpu.make_async_copy(dstr.at[s].at[pl.ds(k * _CH, _CH)],
                              dcp, isems[p]).wait()

        def irow(w, _):
            for j in range(_W // 16):
                v = scp[w, pl.ds(j * 16, 16)]
                scp[w, pl.ds(j * 16, 16)] = v * 2 + c
            return 0
        lax.fori_loop(0, _CH, irow, 0, unroll=2)

        if k + 1 < _NCHUNK:
            fetch_chunk(k + 1, 1 - p)

        pltpu.async_copy(h2.at[scp.at[0]], gb0, gs0)
        pltpu.async_copy(h2.at[scp.at[1]], gb1, gs1)

        def step(i, _):
            g = i * 2
            for b in range(2):
                w = g + b
                gbuf, gsem, ssem = gbufs[b], gsems[b], ssems[b]
                pltpu.make_async_copy(h2.at[scp.at[w]], gbuf, gsem).wait()
                pltpu.async_copy(gbuf, acc.at[dcp.at[w]], ssem, add=True)
                pltpu.make_async_copy(gbuf, acc.at[dcp.at[w]], ssem).wait()

                @pl.when(w + 2 < _CH)
                def _():
                    pltpu.async_copy(h2.at[scp.at[w + 2]], gbuf, gsem)
            return 0
        lax.fori_loop(0, _CH // 2, step, 0)

    plsc.subcore_barrier()

    rbase = s * _RPT
    pltpu.sync_copy(acc.at[pl.ds(rbase, _RPT)],
                    out.at[c].at[pl.ds(rbase, _RPT)])


def _build_agg(tw):
    mesh = plsc.VectorSubcoreMesh(core_axis_name="c", subcore_axis_name="s")
    return pl.kernel(
        functools.partial(_agg_body, tw),
        out_type=jax.ShapeDtypeStruct((_NC, _NACC, tw), jnp.float32),
        mesh=mesh,
        scratch_types=[
            pltpu.VMEM_SHARED((_NACC, tw), jnp.float32),
            pltpu.VMEM((2, _CH, _W), jnp.int32),
            pltpu.VMEM((2, _CH, _W), jnp.int32),
            pltpu.VMEM((_W, tw), jnp.float32),
            pltpu.VMEM((_W, tw), jnp.float32),
        ] + [pltpu.SemaphoreType.DMA] * 6)


_agg = _build_agg(_HF)

_WC = 128
_NWC = _EPAD // _NS // _WC
_CHC = 8


def _cnt_body(dstr, out, acc, dst_c, ones_v, ss0, ss1, ss2, ss3, is0, is1):
    c = lax.axis_index("c")
    s = lax.axis_index("s")
    isems = (is0, is1)
    ssems = (ss0, ss1, ss2, ss3)
    z16 = jnp.zeros((16,), jnp.float32)
    o16 = jnp.ones((16,), jnp.float32)

    def zrow(i, _):
        for j in range(8):
            ones_v[i, pl.ds(j * 16, 16)] = z16
        return 0
    lax.fori_loop(0, _WC, zrow, 0, unroll=2)
    zbase = s * (_NACC // _NS)
    nz = _NACC // _NS // _WC
    for k in range(nz):
        pltpu.async_copy(ones_v, acc.at[pl.ds(zbase + k * _WC, _WC)],
                         ssems[k % 4])
    for k in range(nz):
        pltpu.make_async_copy(ones_v, acc.at[pl.ds(zbase + k * _WC, _WC)],
                              ssems[k % 4]).wait()

    def orow(i, _):
        for j in range(8):
            ones_v[i, pl.ds(j * 16, 16)] = o16
        return 0
    lax.fori_loop(0, _WC, orow, 0, unroll=2)

    plsc.subcore_barrier()

    def fetch_chunk(k, p):
        pltpu.async_copy(dstr.at[s].at[pl.ds(k * _CHC, _CHC)],
                         dst_c.at[p], isems[p])

    fetch_chunk(0, 0)
    for k in range(_NWC // _CHC):
        p = k % 2
        dcp = dst_c.at[p]
        pltpu.make_async_copy(dstr.at[s].at[pl.ds(k * _CHC, _CHC)],
                              dcp, isems[p]).wait()
        if k + 1 < _NWC // _CHC:
            fetch_chunk(k + 1, 1 - p)

        for w in range(_CHC):
            if w >= 4:
                pltpu.make_async_copy(ones_v, acc.at[dcp.at[w - 4]],
                                      ssems[w % 4]).wait()
            pltpu.async_copy(ones_v, acc.at[dcp.at[w]], ssems[w % 4],
                             add=True)
        for w in range(_CHC - 4, _CHC):
            pltpu.make_async_copy(ones_v, acc.at[dcp.at[w]],
                                  ssems[w % 4]).wait()

    plsc.subcore_barrier()
    rbase = s * _RPT
    pltpu.sync_copy(acc.at[pl.ds(rbase, _RPT)],
                    out.at[c].at[pl.ds(rbase, _RPT)])


def _build_cnt():
    mesh = plsc.VectorSubcoreMesh(core_axis_name="c", subcore_axis_name="s")
    return pl.kernel(
        _cnt_body,
        out_type=jax.ShapeDtypeStruct((_NC, _NACC, _HF), jnp.float32),
        mesh=mesh,
        scratch_types=[
            pltpu.VMEM_SHARED((_NACC, _HF), jnp.float32),
            pltpu.VMEM((2, _CHC, _WC), jnp.int32),
            pltpu.VMEM((_WC, _HF), jnp.float32),
        ] + [pltpu.SemaphoreType.DMA] * 6)


_cnt = _build_cnt()


def _row_blocked(kern, n, h, extra_specs, out_cols):
    return pl.pallas_call(
        kern,
        grid=(n // _BN,),
        in_specs=[pl.BlockSpec((_BN, h), lambda i: (i, 0))] + extra_specs,
        out_specs=pl.BlockSpec((_BN, out_cols), lambda i: (i, 0)),
        out_shape=jax.ShapeDtypeStruct((n, out_cols), jnp.float32),
    )


def kernel(x, edge_index, W_in, b_in, Wl, bl, Wr, gamma, beta, W_out, b_out):
    n, in_dim = x.shape
    h_dim = W_in.shape[0]
    L = Wl.shape[0]
    e = edge_index.shape[1]
    src = edge_index[0]
    dst = edge_index[1]

    pad = _EPAD - e
    srcr = jnp.concatenate([src, jnp.zeros((pad,), jnp.int32)]
                           ).reshape(_NS, _NWIN, _W)
    dstr = jnp.concatenate([dst, jnp.full((pad,), _N, jnp.int32)]
                           ).reshape(_NS, _NWIN, _W)

    wspec = pl.BlockSpec((h_dim, in_dim), lambda i: (0, 0))
    vspec = pl.BlockSpec((1, h_dim), lambda i: (0, 0))

    h = _row_blocked(_in_proj_kernel, n, in_dim,
                     [wspec, vspec], h_dim)(x, W_in, b_in.reshape(1, h_dim))

    layer = pl.pallas_call(
        _layer_kernel,
        grid=(n // _BN,),
        in_specs=[
            pl.BlockSpec((_BN, _HF), lambda i: (i, 0)),
            pl.BlockSpec((_BN, _HF), lambda i: (i, 0)),
            pl.BlockSpec((_BN, _HF), lambda i: (i, 0)),
            pl.BlockSpec((_BN, h_dim), lambda i: (i, 0)),
            pl.BlockSpec((h_dim, h_dim), lambda i: (0, 0)),
            vspec,
            pl.BlockSpec((h_dim, h_dim), lambda i: (0, 0)),
            vspec, vspec,
        ],
        out_specs=pl.BlockSpec((_BN, h_dim), lambda i: (i, 0)),
        out_shape=jax.ShapeDtypeStruct((n, h_dim), jnp.float32),
    )

    cnt16 = _cnt(dstr.reshape(_NS, _NWC, _WC))[0]
    for i in range(L):
        h2 = h.reshape(2 * n, _HF)
        aggs = _agg(h2, srcr, dstr)
        h = layer(aggs[0], aggs[1], cnt16, h, Wl[i],
                  bl[i].reshape(1, h_dim), Wr[i],
                  gamma[i].reshape(1, h_dim), beta[i].reshape(1, h_dim))

    out = _row_blocked(_out_proj_kernel, n, h_dim,
                       [vspec, pl.BlockSpec((1, 1), lambda i: (0, 0))], 1)(
        h, W_out.reshape(1, h_dim), b_out.reshape(1, 1))
    return out.reshape(-1)
